# Initial kernel scaffold; baseline (speedup 1.0000x reference)
import jax
import jax.numpy as jnp
from jax.experimental import pallas as pl


def kernel(p, labels, nsample):
    raise NotImplementedError("write your pallas kernel here")


# speedup vs baseline: 5.5306x; 5.5306x over previous
import functools

import jax
import jax.numpy as jnp
from jax import lax
from jax.experimental import pallas as pl
from jax.experimental.pallas import tpu as pltpu
from jax.experimental.pallas import tpu_sc as plsc

N = 8192
K = 16
NS = K - 1
C = 13
R = 256
NCH = 16
CW = N // NCH
GRID = N // R

_NCORES = 2
_NSUB = 16
_NW = _NCORES * _NSUB
_PPW = N // _NW
_GPW = _PPW // 16


def _knn_body(p_blk, pTc, lab, delta_ref, dist_ref, idx_ref, cls_ref, acc_ref):
    delta = delta_ref[0, 0]
    xr = p_blk[:, 0:1]
    yr = p_blk[:, 1:2]
    zr = p_blk[:, 2:3]

    def dist_chunk(c, carry):
        pc = pTc[c]
        dx = xr - pc[0:1, :]
        dy = yr - pc[1:2, :]
        dz = zr - pc[2:3, :]
        acc_ref[c] = dx * dx + dy * dy + dz * dz + delta
        return carry

    lax.fori_loop(0, NCH, dist_chunk, 0)

    inf = jnp.float32(jnp.inf)
    big = jnp.int32(2 * N)
    mprev = jnp.full((R, 1), -inf, jnp.float32)
    sprev = jnp.full((R, 1), -1, jnp.int32)
    for t in range(K):
        def scan_chunk(c, carry, mp=mprev, sp=sprev):
            m, s = carry
            a = acc_ref[c]
            iota = lax.broadcasted_iota(jnp.int32, (R, CW), 1) + c * CW
            keep = (a > mp) | ((a == mp) & (iota > sp))
            am = jnp.where(keep, a, inf)
            cm = jnp.min(am, axis=1, keepdims=True)
            cs = jnp.min(jnp.where(am == cm, iota, big), axis=1, keepdims=True)
            take = (cm < m) | ((cm == m) & (cs < s))
            return (jnp.where(take, cm, m), jnp.where(take, cs, s))

        m, s = lax.fori_loop(
            0, NCH, scan_chunk,
            (jnp.full((R, 1), inf, jnp.float32), jnp.full((R, 1), big, jnp.int32)))
        dist_ref[:, t:t + 1] = jnp.sqrt(jnp.maximum(m, 1e-12))
        idx_ref[:, t:t + 1] = s
        mprev, sprev = m, s

    lv = lab[...]
    mx = jnp.max(lv, axis=1, keepdims=True)
    i13 = lax.broadcasted_iota(jnp.int32, lv.shape, 1)
    cl = jnp.min(jnp.where(lv == mx, i13, jnp.int32(C)), axis=1, keepdims=True)
    cls_ref[...] = cl.reshape(1, R, 1)


_knn = pl.pallas_call(
    _knn_body,
    grid=(GRID,),
    in_specs=[
        pl.BlockSpec((R, 3), lambda b: (b, 0)),
        pl.BlockSpec((NCH, 3, CW), lambda b: (0, 0, 0)),
        pl.BlockSpec((R, C), lambda b: (b, 0)),
        pl.BlockSpec((1, 1), lambda b: (0, 0)),
    ],
    out_specs=[
        pl.BlockSpec((R, K), lambda b: (b, 0)),
        pl.BlockSpec((R, K), lambda b: (b, 0)),
        pl.BlockSpec((1, R, 1), lambda b: (b, 0, 0)),
    ],
    out_shape=[
        jax.ShapeDtypeStruct((N, K), jnp.float32),
        jax.ShapeDtypeStruct((N, K), jnp.int32),
        jax.ShapeDtypeStruct((GRID, R, 1), jnp.int32),
    ],
    scratch_shapes=[pltpu.VMEM((NCH, R, CW), jnp.float32)],
)


def _sc_body(idxT_hbm, distT_hbm, cls_hbm, out_hbm, cls_v, idx_v, dist_v, out_v):
    wid = lax.axis_index("s") * _NCORES + lax.axis_index("c")
    base = wid * _PPW
    pltpu.sync_copy(cls_hbm, cls_v)
    pltpu.sync_copy(idxT_hbm.at[:, pl.ds(base, _PPW)], idx_v)
    pltpu.sync_copy(distT_hbm.at[:, pl.ds(base, _PPW)], dist_v)

    def grp(g, carry):
        own = cls_v[pl.ds(base + g * 16, 16)]
        cnt = jnp.zeros((16,), jnp.int32)
        wsum = jnp.zeros((16,), jnp.float32)
        wneg = jnp.zeros((16,), jnp.float32)
        for j in range(1, K):
            idxv = idx_v[j, pl.ds(g * 16, 16)]
            nlab = plsc.load_gather(cls_v, [idxv])
            dv = dist_v[j, pl.ds(g * 16, 16)]
            wv = jnp.exp(-dv)
            agree = nlab == own
            cnt = cnt + agree.astype(jnp.int32)
            wsum = wsum + wv
            wneg = wneg + jnp.where(agree, jnp.zeros((16,), jnp.float32), wv)
        amb = wneg / (wsum + 1e-8)
        res = jnp.where(cnt == 0, jnp.full((16,), 1.0, jnp.float32),
                        jnp.where(cnt == NS, jnp.zeros((16,), jnp.float32), amb))
        out_v[pl.ds(g * 16, 16)] = res
        return carry

    lax.fori_loop(0, _GPW, grp, 0)
    pltpu.sync_copy(out_v, out_hbm.at[pl.ds(base, _PPW)])


@functools.lru_cache(maxsize=1)
def _sc_ambiguity():
    return functools.partial(
        pl.kernel,
        mesh=plsc.VectorSubcoreMesh(core_axis_name="c", subcore_axis_name="s"),
        out_type=jax.ShapeDtypeStruct((N,), jnp.float32),
        scratch_types=[
            pltpu.VMEM((N,), jnp.int32),
            pltpu.VMEM((K, _PPW), jnp.int32),
            pltpu.VMEM((K, _PPW), jnp.float32),
            pltpu.VMEM((_PPW,), jnp.float32),
        ],
        compiler_params=pltpu.CompilerParams(needs_layout_passes=False),
    )(_sc_body)


def kernel(p, labels, nsample):
    delta = (jnp.asarray(nsample, jnp.float32) - jnp.float32(K)).reshape(1, 1)
    pTc = p.T.reshape(3, NCH, CW).transpose(1, 0, 2)
    dist, idx, cls3 = _knn(p, pTc, labels, delta)
    cls = cls3.reshape(N)
    return _sc_ambiguity()(idx.T, dist.T, cls)


# speedup vs baseline: 6.7316x; 1.2172x over previous
import functools

import jax
import jax.numpy as jnp
from jax import lax
from jax.experimental import pallas as pl
from jax.experimental.pallas import tpu as pltpu
from jax.experimental.pallas import tpu_sc as plsc

N = 8192
K = 16
NS = K - 1
C = 13
R = 256
NCH = 16
CW = N // NCH
GRID = N // R

_NCORES = 2
_NSUB = 16
_NW = _NCORES * _NSUB
_PPW = N // _NW
_GPW = _PPW // 16


def _knn_body(p_blk, pTc, lab, delta_ref, dist_ref, idx_ref, cls_ref, acc_ref):
    delta = delta_ref[0, 0]
    xr = p_blk[:, 0:1]
    yr = p_blk[:, 1:2]
    zr = p_blk[:, 2:3]

    inf = jnp.float32(jnp.inf)
    big = jnp.int32(2 * N)
    iota = lax.broadcasted_iota(jnp.int32, (R, CW), 1)
    init = (jnp.full((R, 1), inf, jnp.float32), jnp.full((R, 1), big, jnp.int32))

    def merge(carry, cm, cs):
        m, s = carry
        take = (cm < m) | ((cm == m) & (cs < s))
        return (jnp.where(take, cm, m), jnp.where(take, cs, s))

    def dist_chunk(c, carry):
        pc = pTc[c]
        dx = xr - pc[0:1, :]
        dy = yr - pc[1:2, :]
        dz = zr - pc[2:3, :]
        a = dx * dx + dy * dy + dz * dz + delta
        acc_ref[c] = a
        cm = jnp.min(a, axis=1, keepdims=True)
        cs = jnp.min(jnp.where(a == cm, iota, big), axis=1, keepdims=True) + c * CW
        return merge(carry, cm, cs)

    m, s = lax.fori_loop(0, NCH, dist_chunk, init)
    dist_ref[:, 0:1] = jnp.sqrt(jnp.maximum(m, 1e-12))
    idx_ref[:, 0:1] = s

    sprev = s
    for t in range(1, K):
        def scan_chunk(c, carry, sp=sprev):
            a = jnp.where(iota == sp - c * CW, inf, acc_ref[c])
            acc_ref[c] = a
            cm = jnp.min(a, axis=1, keepdims=True)
            cs = jnp.min(jnp.where(a == cm, iota, big), axis=1, keepdims=True) + c * CW
            return merge(carry, cm, cs)

        m, s = lax.fori_loop(0, NCH, scan_chunk, init)
        dist_ref[:, t:t + 1] = jnp.sqrt(jnp.maximum(m, 1e-12))
        idx_ref[:, t:t + 1] = s
        sprev = s

    lv = lab[...]
    mx = jnp.max(lv, axis=1, keepdims=True)
    i13 = lax.broadcasted_iota(jnp.int32, lv.shape, 1)
    cl = jnp.min(jnp.where(lv == mx, i13, jnp.int32(C)), axis=1, keepdims=True)
    cls_ref[...] = cl.reshape(1, R, 1)


_knn = pl.pallas_call(
    _knn_body,
    grid=(GRID,),
    in_specs=[
        pl.BlockSpec((R, 3), lambda b: (b, 0)),
        pl.BlockSpec((NCH, 3, CW), lambda b: (0, 0, 0)),
        pl.BlockSpec((R, C), lambda b: (b, 0)),
        pl.BlockSpec((1, 1), lambda b: (0, 0)),
    ],
    out_specs=[
        pl.BlockSpec((R, K), lambda b: (b, 0)),
        pl.BlockSpec((R, K), lambda b: (b, 0)),
        pl.BlockSpec((1, R, 1), lambda b: (b, 0, 0)),
    ],
    out_shape=[
        jax.ShapeDtypeStruct((N, K), jnp.float32),
        jax.ShapeDtypeStruct((N, K), jnp.int32),
        jax.ShapeDtypeStruct((GRID, R, 1), jnp.int32),
    ],
    scratch_shapes=[pltpu.VMEM((NCH, R, CW), jnp.float32)],
)


def _sc_body(idxT_hbm, distT_hbm, cls_hbm, out_hbm, cls_v, idx_v, dist_v, out_v):
    wid = lax.axis_index("s") * _NCORES + lax.axis_index("c")
    base = wid * _PPW
    pltpu.sync_copy(cls_hbm, cls_v)
    pltpu.sync_copy(idxT_hbm.at[:, pl.ds(base, _PPW)], idx_v)
    pltpu.sync_copy(distT_hbm.at[:, pl.ds(base, _PPW)], dist_v)

    def grp(g, carry):
        own = cls_v[pl.ds(base + g * 16, 16)]
        cnt = jnp.zeros((16,), jnp.int32)
        wsum = jnp.zeros((16,), jnp.float32)
        wneg = jnp.zeros((16,), jnp.float32)
        for j in range(1, K):
            idxv = idx_v[j, pl.ds(g * 16, 16)]
            nlab = plsc.load_gather(cls_v, [idxv])
            dv = dist_v[j, pl.ds(g * 16, 16)]
            wv = jnp.exp(-dv)
            agree = nlab == own
            cnt = cnt + agree.astype(jnp.int32)
            wsum = wsum + wv
            wneg = wneg + jnp.where(agree, jnp.zeros((16,), jnp.float32), wv)
        amb = wneg / (wsum + 1e-8)
        res = jnp.where(cnt == 0, jnp.full((16,), 1.0, jnp.float32),
                        jnp.where(cnt == NS, jnp.zeros((16,), jnp.float32), amb))
        out_v[pl.ds(g * 16, 16)] = res
        return carry

    lax.fori_loop(0, _GPW, grp, 0)
    pltpu.sync_copy(out_v, out_hbm.at[pl.ds(base, _PPW)])


@functools.lru_cache(maxsize=1)
def _sc_ambiguity():
    return functools.partial(
        pl.kernel,
        mesh=plsc.VectorSubcoreMesh(core_axis_name="c", subcore_axis_name="s"),
        out_type=jax.ShapeDtypeStruct((N,), jnp.float32),
        scratch_types=[
            pltpu.VMEM((N,), jnp.int32),
            pltpu.VMEM((K, _PPW), jnp.int32),
            pltpu.VMEM((K, _PPW), jnp.float32),
            pltpu.VMEM((_PPW,), jnp.float32),
        ],
        compiler_params=pltpu.CompilerParams(needs_layout_passes=False),
    )(_sc_body)


def kernel(p, labels, nsample):
    delta = (jnp.asarray(nsample, jnp.float32) - jnp.float32(K)).reshape(1, 1)
    pTc = p.T.reshape(3, NCH, CW).transpose(1, 0, 2)
    dist, idx, cls3 = _knn(p, pTc, labels, delta)
    cls = cls3.reshape(N)
    return _sc_ambiguity()(idx.T, dist.T, cls)


# speedup vs baseline: 13.9060x; 2.0658x over previous
import functools

import jax
import jax.numpy as jnp
from jax import lax
from jax.experimental import pallas as pl
from jax.experimental.pallas import tpu as pltpu
from jax.experimental.pallas import tpu_sc as plsc

N = 8192
K = 16
NS = K - 1
C = 13
R = 256
NCH = 8
CW = N // NCH
GRID = N // R

_NCORES = 2
_NSUB = 16
_NW = _NCORES * _NSUB
_PPW = N // _NW
_GPW = _PPW // 16


def _knn_body(p_blk, pTc, lab, delta_ref, dist_ref, idx_ref, cls_ref, acc_ref):
    delta = delta_ref[0, 0]
    xr = p_blk[:, 0:1]
    yr = p_blk[:, 1:2]
    zr = p_blk[:, 2:3]

    inf = jnp.float32(jnp.inf)
    big = jnp.float32(2 * N)
    iota = lax.broadcasted_iota(jnp.int32, (1, CW), 1).astype(jnp.float32)
    init = (jnp.full((R, 1), inf, jnp.float32), jnp.full((R, 1), big, jnp.float32))

    def merge(carry, cm, cs):
        m, s = carry
        take = (cm < m) | ((cm == m) & (cs < s))
        return (jnp.where(take, cm, m), jnp.where(take, cs, s))

    def dist_chunk(c, carry):
        pc = pTc[c]
        dx = xr - pc[0:1, :]
        dy = yr - pc[1:2, :]
        dz = zr - pc[2:3, :]
        a = dx * dx + dy * dy + dz * dz + delta
        acc_ref[c] = a
        cm = jnp.min(a, axis=1, keepdims=True)
        cf = (c * CW).astype(jnp.float32)
        cs = jnp.min(jnp.where(a == cm, iota, big), axis=1, keepdims=True) + cf
        return merge(carry, cm, cs)

    m, s = lax.fori_loop(0, NCH, dist_chunk, init)
    dist_ref[:, 0:1] = jnp.sqrt(jnp.maximum(m, 1e-12))
    idx_ref[:, 0:1] = s.astype(jnp.int32)

    sprev = s
    for t in range(1, K):
        def scan_chunk(c, carry, sp=sprev):
            cf = (c * CW).astype(jnp.float32)
            a = jnp.where(iota == sp - cf, inf, acc_ref[c])
            acc_ref[c] = a
            cm = jnp.min(a, axis=1, keepdims=True)
            cs = jnp.min(jnp.where(a == cm, iota, big), axis=1, keepdims=True) + cf
            return merge(carry, cm, cs)

        m, s = lax.fori_loop(0, NCH, scan_chunk, init)
        dist_ref[:, t:t + 1] = jnp.sqrt(jnp.maximum(m, 1e-12))
        idx_ref[:, t:t + 1] = s.astype(jnp.int32)
        sprev = s

    lv = lab[...]
    mx = jnp.max(lv, axis=1, keepdims=True)
    i13 = lax.broadcasted_iota(jnp.int32, lv.shape, 1)
    cl = jnp.min(jnp.where(lv == mx, i13, jnp.int32(C)), axis=1, keepdims=True)
    cls_ref[...] = cl.reshape(1, R, 1)


_knn = pl.pallas_call(
    _knn_body,
    grid=(GRID,),
    in_specs=[
        pl.BlockSpec((R, 3), lambda b: (b, 0)),
        pl.BlockSpec((NCH, 3, CW), lambda b: (0, 0, 0)),
        pl.BlockSpec((R, C), lambda b: (b, 0)),
        pl.BlockSpec((1, 1), lambda b: (0, 0)),
    ],
    out_specs=[
        pl.BlockSpec((R, K), lambda b: (b, 0)),
        pl.BlockSpec((R, K), lambda b: (b, 0)),
        pl.BlockSpec((1, R, 1), lambda b: (b, 0, 0)),
    ],
    out_shape=[
        jax.ShapeDtypeStruct((N, K), jnp.float32),
        jax.ShapeDtypeStruct((N, K), jnp.int32),
        jax.ShapeDtypeStruct((GRID, R, 1), jnp.int32),
    ],
    scratch_shapes=[pltpu.VMEM((NCH, R, CW), jnp.float32)],
)


def _sc_body(idxT_hbm, distT_hbm, cls_hbm, out_hbm, cls_v, idx_v, dist_v, out_v):
    wid = lax.axis_index("s") * _NCORES + lax.axis_index("c")
    base = wid * _PPW
    pltpu.sync_copy(cls_hbm, cls_v)
    pltpu.sync_copy(idxT_hbm.at[:, pl.ds(base, _PPW)], idx_v)
    pltpu.sync_copy(distT_hbm.at[:, pl.ds(base, _PPW)], dist_v)

    def grp(g, carry):
        own = cls_v[pl.ds(base + g * 16, 16)]
        cnt = jnp.zeros((16,), jnp.int32)
        wsum = jnp.zeros((16,), jnp.float32)
        wneg = jnp.zeros((16,), jnp.float32)
        for j in range(1, K):
            idxv = idx_v[j, pl.ds(g * 16, 16)]
            nlab = plsc.load_gather(cls_v, [idxv])
            dv = dist_v[j, pl.ds(g * 16, 16)]
            wv = jnp.exp(-dv)
            agree = nlab == own
            cnt = cnt + agree.astype(jnp.int32)
            wsum = wsum + wv
            wneg = wneg + jnp.where(agree, jnp.zeros((16,), jnp.float32), wv)
        amb = wneg / (wsum + 1e-8)
        res = jnp.where(cnt == 0, jnp.full((16,), 1.0, jnp.float32),
                        jnp.where(cnt == NS, jnp.zeros((16,), jnp.float32), amb))
        out_v[pl.ds(g * 16, 16)] = res
        return carry

    lax.fori_loop(0, _GPW, grp, 0)
    pltpu.sync_copy(out_v, out_hbm.at[pl.ds(base, _PPW)])


@functools.lru_cache(maxsize=1)
def _sc_ambiguity():
    return functools.partial(
        pl.kernel,
        mesh=plsc.VectorSubcoreMesh(core_axis_name="c", subcore_axis_name="s"),
        out_type=jax.ShapeDtypeStruct((N,), jnp.float32),
        scratch_types=[
            pltpu.VMEM((N,), jnp.int32),
            pltpu.VMEM((K, _PPW), jnp.int32),
            pltpu.VMEM((K, _PPW), jnp.float32),
            pltpu.VMEM((_PPW,), jnp.float32),
        ],
        compiler_params=pltpu.CompilerParams(needs_layout_passes=False),
    )(_sc_body)


def kernel(p, labels, nsample):
    delta = (jnp.asarray(nsample, jnp.float32) - jnp.float32(K)).reshape(1, 1)
    pTc = p.T.reshape(3, NCH, CW).transpose(1, 0, 2)
    dist, idx, cls3 = _knn(p, pTc, labels, delta)
    cls = cls3.reshape(N)
    return _sc_ambiguity()(idx.T, dist.T, cls)


# speedup vs baseline: 17.9184x; 1.2885x over previous
import functools

import jax
import jax.numpy as jnp
from jax import lax
from jax.experimental import pallas as pl
from jax.experimental.pallas import tpu as pltpu
from jax.experimental.pallas import tpu_sc as plsc

N = 8192
K = 16
NS = K - 1
C = 13
R = 256
NCH = 8
CW = N // NCH
NCHP = NCH // 2
GRID = N // R

_NCORES = 2
_NSUB = 16
_NW = _NCORES * _NSUB
_PPW = N // _NW
_GPW = _PPW // 16


def _knn_body(p_blk, pTc, lab, delta_ref, dist_ref, idx_ref, cls_ref,
              pm_ref, pa_ref, px_ref):
    delta = delta_ref[0, 0]
    xr = p_blk[:, 0:1]
    yr = p_blk[:, 1:2]
    zr = p_blk[:, 2:3]

    inf = jnp.float32(jnp.inf)
    big = jnp.float32(4 * N)
    half = jnp.float32(N // 2)
    iota = lax.broadcasted_iota(jnp.int32, (1, CW), 1).astype(jnp.float32)
    init = (jnp.full((R, 1), inf, jnp.float32), jnp.full((R, 1), big, jnp.float32))

    def merge(carry, cm, cs):
        m, s = carry
        take = (cm < m) | ((cm == m) & (cs < s))
        return (jnp.where(take, cm, m), jnp.where(take, cs, s))

    def dists(ci):
        pc = pTc[ci]
        dx = xr - pc[0:1, :]
        dy = yr - pc[1:2, :]
        dz = zr - pc[2:3, :]
        return dx * dx + dy * dy + dz * dz + delta

    def dist_chunk(c, carry):
        alo = dists(c)
        ahi = dists(c + NCHP)
        islo = alo <= ahi
        pm = jnp.where(islo, alo, ahi)
        px = jnp.where(islo, ahi, alo)
        cf = (c * CW).astype(jnp.float32)
        pa = iota + (cf + jnp.where(islo, jnp.float32(0), half))
        pm_ref[c] = pm
        pa_ref[c] = pa
        px_ref[c] = px
        cm = jnp.min(pm, axis=1, keepdims=True)
        cs = jnp.min(jnp.where(pm == cm, pa, big), axis=1, keepdims=True)
        return merge(carry, cm, cs)

    m, s = lax.fori_loop(0, NCHP, dist_chunk, init)
    dist_ref[:, 0:1] = jnp.sqrt(jnp.maximum(m, 1e-12))
    idx_ref[:, 0:1] = s.astype(jnp.int32)

    sprev = s
    for t in range(1, K):
        ge = sprev >= half
        pid = jnp.where(ge, sprev - half, sprev)
        sib = jnp.where(ge, sprev - half, sprev + half)

        def scan_chunk(c, carry, pid=pid, sib=sib):
            eqp = iota == (pid - (c * CW).astype(jnp.float32))
            pm2 = jnp.where(eqp, px_ref[c], pm_ref[c])
            pa2 = jnp.where(eqp, sib, pa_ref[c])
            px_ref[c] = jnp.where(eqp, inf, px_ref[c])
            pm_ref[c] = pm2
            pa_ref[c] = pa2
            cm = jnp.min(pm2, axis=1, keepdims=True)
            cs = jnp.min(jnp.where(pm2 == cm, pa2, big), axis=1, keepdims=True)
            return merge(carry, cm, cs)

        m, s = lax.fori_loop(0, NCHP, scan_chunk, init)
        dist_ref[:, t:t + 1] = jnp.sqrt(jnp.maximum(m, 1e-12))
        idx_ref[:, t:t + 1] = s.astype(jnp.int32)
        sprev = s

    lv = lab[...]
    mx = jnp.max(lv, axis=1, keepdims=True)
    i13 = lax.broadcasted_iota(jnp.int32, lv.shape, 1)
    cl = jnp.min(jnp.where(lv == mx, i13, jnp.int32(C)), axis=1, keepdims=True)
    cls_ref[...] = cl.reshape(1, R, 1)


def _build_knn(interpret=False):
    return pl.pallas_call(
        _knn_body,
        grid=(GRID,),
        in_specs=[
            pl.BlockSpec((R, 3), lambda b: (b, 0)),
            pl.BlockSpec((NCH, 3, CW), lambda b: (0, 0, 0)),
            pl.BlockSpec((R, C), lambda b: (b, 0)),
            pl.BlockSpec((1, 1), lambda b: (0, 0)),
        ],
        out_specs=[
            pl.BlockSpec((R, K), lambda b: (b, 0)),
            pl.BlockSpec((R, K), lambda b: (b, 0)),
            pl.BlockSpec((1, R, 1), lambda b: (b, 0, 0)),
        ],
        out_shape=[
            jax.ShapeDtypeStruct((N, K), jnp.float32),
            jax.ShapeDtypeStruct((N, K), jnp.int32),
            jax.ShapeDtypeStruct((GRID, R, 1), jnp.int32),
        ],
        scratch_shapes=[
            pltpu.VMEM((NCHP, R, CW), jnp.float32),
            pltpu.VMEM((NCHP, R, CW), jnp.float32),
            pltpu.VMEM((NCHP, R, CW), jnp.float32),
        ],
        interpret=interpret,
    )


_knn = _build_knn()


def _sc_body(idxT_hbm, distT_hbm, cls_hbm, out_hbm, cls_v, idx_v, dist_v, out_v):
    wid = lax.axis_index("s") * _NCORES + lax.axis_index("c")
    base = wid * _PPW
    pltpu.sync_copy(cls_hbm, cls_v)
    pltpu.sync_copy(idxT_hbm.at[:, pl.ds(base, _PPW)], idx_v)
    pltpu.sync_copy(distT_hbm.at[:, pl.ds(base, _PPW)], dist_v)

    def grp(g, carry):
        own = cls_v[pl.ds(base + g * 16, 16)]
        cnt = jnp.zeros((16,), jnp.int32)
        wsum = jnp.zeros((16,), jnp.float32)
        wneg = jnp.zeros((16,), jnp.float32)
        for j in range(1, K):
            idxv = idx_v[j, pl.ds(g * 16, 16)]
            nlab = plsc.load_gather(cls_v, [idxv])
            dv = dist_v[j, pl.ds(g * 16, 16)]
            wv = jnp.exp(-dv)
            agree = nlab == own
            cnt = cnt + agree.astype(jnp.int32)
            wsum = wsum + wv
            wneg = wneg + jnp.where(agree, jnp.zeros((16,), jnp.float32), wv)
        amb = wneg / (wsum + 1e-8)
        res = jnp.where(cnt == 0, jnp.full((16,), 1.0, jnp.float32),
                        jnp.where(cnt == NS, jnp.zeros((16,), jnp.float32), amb))
        out_v[pl.ds(g * 16, 16)] = res
        return carry

    lax.fori_loop(0, _GPW, grp, 0)
    pltpu.sync_copy(out_v, out_hbm.at[pl.ds(base, _PPW)])


@functools.lru_cache(maxsize=1)
def _sc_ambiguity():
    return functools.partial(
        pl.kernel,
        mesh=plsc.VectorSubcoreMesh(core_axis_name="c", subcore_axis_name="s"),
        out_type=jax.ShapeDtypeStruct((N,), jnp.float32),
        scratch_types=[
            pltpu.VMEM((N,), jnp.int32),
            pltpu.VMEM((K, _PPW), jnp.int32),
            pltpu.VMEM((K, _PPW), jnp.float32),
            pltpu.VMEM((_PPW,), jnp.float32),
        ],
        compiler_params=pltpu.CompilerParams(needs_layout_passes=False),
    )(_sc_body)


def kernel(p, labels, nsample):
    delta = (jnp.asarray(nsample, jnp.float32) - jnp.float32(K)).reshape(1, 1)
    pTc = p.T.reshape(3, NCH, CW).transpose(1, 0, 2)
    dist, idx, cls3 = _knn(p, pTc, labels, delta)
    cls = cls3.reshape(N)
    return _sc_ambiguity()(idx.T, dist.T, cls)


# speedup vs baseline: 19.0847x; 1.0651x over previous
import functools

import jax
import jax.numpy as jnp
from jax import lax
from jax.experimental import pallas as pl
from jax.experimental.pallas import tpu as pltpu
from jax.experimental.pallas import tpu_sc as plsc

N = 8192
K = 16
NS = K - 1
C = 13
R = 256
NCH = 4
CW = N // NCH
NCHP = NCH // 2
GRID = N // R

_NCORES = 2
_NSUB = 16
_NW = _NCORES * _NSUB
_PPW = N // _NW
_GPW = _PPW // 16


def _knn_body(p_blk, pTc, lab, delta_ref, dist_ref, idx_ref, cls_ref,
              pm_ref, pa_ref, px_ref):
    delta = delta_ref[0, 0]
    xr = p_blk[:, 0:1]
    yr = p_blk[:, 1:2]
    zr = p_blk[:, 2:3]

    inf = jnp.float32(jnp.inf)
    big = jnp.float32(4 * N)
    half = jnp.float32(N // 2)
    iota = lax.broadcasted_iota(jnp.int32, (1, CW), 1).astype(jnp.float32)
    init = (jnp.full((R, 1), inf, jnp.float32), jnp.full((R, 1), big, jnp.float32))

    def merge(carry, cm, cs):
        m, s = carry
        take = (cm < m) | ((cm == m) & (cs < s))
        return (jnp.where(take, cm, m), jnp.where(take, cs, s))

    def dists(ci):
        pc = pTc[ci]
        dx = xr - pc[0:1, :]
        dy = yr - pc[1:2, :]
        dz = zr - pc[2:3, :]
        return dx * dx + dy * dy + dz * dz + delta

    def dist_chunk(c, carry):
        alo = dists(c)
        ahi = dists(c + NCHP)
        islo = alo <= ahi
        pm = jnp.where(islo, alo, ahi)
        px = jnp.where(islo, ahi, alo)
        cf = (c * CW).astype(jnp.float32)
        pa = iota + (cf + jnp.where(islo, jnp.float32(0), half))
        pm_ref[c] = pm
        pa_ref[c] = pa
        px_ref[c] = px
        cm = jnp.min(pm, axis=1, keepdims=True)
        cs = jnp.min(jnp.where(pm == cm, pa, big), axis=1, keepdims=True)
        return merge(carry, cm, cs)

    m, s = lax.fori_loop(0, NCHP, dist_chunk, init)
    dist_ref[:, 0:1] = jnp.sqrt(jnp.maximum(m, 1e-12))
    idx_ref[:, 0:1] = s.astype(jnp.int32)

    sprev = s
    for t in range(1, K):
        ge = sprev >= half
        pid = jnp.where(ge, sprev - half, sprev)
        sib = jnp.where(ge, sprev - half, sprev + half)

        def scan_chunk(c, carry, pid=pid, sib=sib):
            eqp = iota == (pid - (c * CW).astype(jnp.float32))
            pm2 = jnp.where(eqp, px_ref[c], pm_ref[c])
            pa2 = jnp.where(eqp, sib, pa_ref[c])
            px_ref[c] = jnp.where(eqp, inf, px_ref[c])
            pm_ref[c] = pm2
            pa_ref[c] = pa2
            cm = jnp.min(pm2, axis=1, keepdims=True)
            cs = jnp.min(jnp.where(pm2 == cm, pa2, big), axis=1, keepdims=True)
            return merge(carry, cm, cs)

        m, s = lax.fori_loop(0, NCHP, scan_chunk, init)
        dist_ref[:, t:t + 1] = jnp.sqrt(jnp.maximum(m, 1e-12))
        idx_ref[:, t:t + 1] = s.astype(jnp.int32)
        sprev = s

    lv = lab[...]
    mx = jnp.max(lv, axis=1, keepdims=True)
    i13 = lax.broadcasted_iota(jnp.int32, lv.shape, 1)
    cl = jnp.min(jnp.where(lv == mx, i13, jnp.int32(C)), axis=1, keepdims=True)
    cls_ref[...] = cl.reshape(1, R, 1)


def _build_knn(interpret=False):
    return pl.pallas_call(
        _knn_body,
        grid=(GRID,),
        in_specs=[
            pl.BlockSpec((R, 3), lambda b: (b, 0)),
            pl.BlockSpec((NCH, 3, CW), lambda b: (0, 0, 0)),
            pl.BlockSpec((R, C), lambda b: (b, 0)),
            pl.BlockSpec((1, 1), lambda b: (0, 0)),
        ],
        out_specs=[
            pl.BlockSpec((R, K), lambda b: (b, 0)),
            pl.BlockSpec((R, K), lambda b: (b, 0)),
            pl.BlockSpec((1, R, 1), lambda b: (b, 0, 0)),
        ],
        out_shape=[
            jax.ShapeDtypeStruct((N, K), jnp.float32),
            jax.ShapeDtypeStruct((N, K), jnp.int32),
            jax.ShapeDtypeStruct((GRID, R, 1), jnp.int32),
        ],
        scratch_shapes=[
            pltpu.VMEM((NCHP, R, CW), jnp.float32),
            pltpu.VMEM((NCHP, R, CW), jnp.float32),
            pltpu.VMEM((NCHP, R, CW), jnp.float32),
        ],
        interpret=interpret,
    )


_knn = _build_knn()


def _sc_body(idxT_hbm, distT_hbm, cls_hbm, out_hbm, cls_v, idx_v, dist_v, out_v):
    wid = lax.axis_index("s") * _NCORES + lax.axis_index("c")
    base = wid * _PPW
    pltpu.sync_copy(cls_hbm, cls_v)
    pltpu.sync_copy(idxT_hbm.at[:, pl.ds(base, _PPW)], idx_v)
    pltpu.sync_copy(distT_hbm.at[:, pl.ds(base, _PPW)], dist_v)

    def grp(g, carry):
        own = cls_v[pl.ds(base + g * 16, 16)]
        cnt = jnp.zeros((16,), jnp.int32)
        wsum = jnp.zeros((16,), jnp.float32)
        wneg = jnp.zeros((16,), jnp.float32)
        for j in range(1, K):
            idxv = idx_v[j, pl.ds(g * 16, 16)]
            nlab = plsc.load_gather(cls_v, [idxv])
            dv = dist_v[j, pl.ds(g * 16, 16)]
            wv = jnp.exp(-dv)
            agree = nlab == own
            cnt = cnt + agree.astype(jnp.int32)
            wsum = wsum + wv
            wneg = wneg + jnp.where(agree, jnp.zeros((16,), jnp.float32), wv)
        amb = wneg / (wsum + 1e-8)
        res = jnp.where(cnt == 0, jnp.full((16,), 1.0, jnp.float32),
                        jnp.where(cnt == NS, jnp.zeros((16,), jnp.float32), amb))
        out_v[pl.ds(g * 16, 16)] = res
        return carry

    lax.fori_loop(0, _GPW, grp, 0)
    pltpu.sync_copy(out_v, out_hbm.at[pl.ds(base, _PPW)])


@functools.lru_cache(maxsize=1)
def _sc_ambiguity():
    return functools.partial(
        pl.kernel,
        mesh=plsc.VectorSubcoreMesh(core_axis_name="c", subcore_axis_name="s"),
        out_type=jax.ShapeDtypeStruct((N,), jnp.float32),
        scratch_types=[
            pltpu.VMEM((N,), jnp.int32),
            pltpu.VMEM((K, _PPW), jnp.int32),
            pltpu.VMEM((K, _PPW), jnp.float32),
            pltpu.VMEM((_PPW,), jnp.float32),
        ],
        compiler_params=pltpu.CompilerParams(needs_layout_passes=False),
    )(_sc_body)


def kernel(p, labels, nsample):
    delta = (jnp.asarray(nsample, jnp.float32) - jnp.float32(K)).reshape(1, 1)
    pTc = p.T.reshape(3, NCH, CW).transpose(1, 0, 2)
    dist, idx, cls3 = _knn(p, pTc, labels, delta)
    cls = cls3.reshape(N)
    return _sc_ambiguity()(idx.T, dist.T, cls)


# speedup vs baseline: 19.6297x; 1.0286x over previous
import functools

import jax
import jax.numpy as jnp
from jax import lax
from jax.experimental import pallas as pl
from jax.experimental.pallas import tpu as pltpu
from jax.experimental.pallas import tpu_sc as plsc

N = 8192
K = 16
NS = K - 1
C = 13
R = 512
NCH = 4
CW = N // NCH
NCHP = NCH // 2
GRID = N // R

_NCORES = 2
_NSUB = 16
_NW = _NCORES * _NSUB
_PPW = N // _NW
_GPW = _PPW // 16


def _knn_body(p_blk, pTc, lab, delta_ref, dist_ref, idx_ref, cls_ref,
              pm_ref, pa_ref, px_ref):
    delta = delta_ref[0, 0]
    xr = p_blk[:, 0:1]
    yr = p_blk[:, 1:2]
    zr = p_blk[:, 2:3]

    inf = jnp.float32(jnp.inf)
    big = jnp.float32(4 * N)
    half = jnp.float32(N // 2)
    iota = lax.broadcasted_iota(jnp.int32, (1, CW), 1).astype(jnp.float32)
    init = (jnp.full((R, 1), inf, jnp.float32), jnp.full((R, 1), big, jnp.float32))

    def merge(carry, cm, cs):
        m, s = carry
        take = (cm < m) | ((cm == m) & (cs < s))
        return (jnp.where(take, cm, m), jnp.where(take, cs, s))

    def dists(ci):
        pc = pTc[ci]
        dx = xr - pc[0:1, :]
        dy = yr - pc[1:2, :]
        dz = zr - pc[2:3, :]
        return dx * dx + dy * dy + dz * dz + delta

    def dist_chunk(c, carry):
        alo = dists(c)
        ahi = dists(c + NCHP)
        islo = alo <= ahi
        pm = jnp.where(islo, alo, ahi)
        px = jnp.where(islo, ahi, alo)
        cf = (c * CW).astype(jnp.float32)
        pa = iota + (cf + jnp.where(islo, jnp.float32(0), half))
        pm_ref[c] = pm
        pa_ref[c] = pa
        px_ref[c] = px
        cm = jnp.min(pm, axis=1, keepdims=True)
        cs = jnp.min(jnp.where(pm == cm, pa, big), axis=1, keepdims=True)
        return merge(carry, cm, cs)

    m, s = lax.fori_loop(0, NCHP, dist_chunk, init)
    sprev = s
    for t in range(1, K):
        ge = sprev >= half
        pid = jnp.where(ge, sprev - half, sprev)
        sib = jnp.where(ge, sprev - half, sprev + half)

        def scan_chunk(c, carry, pid=pid, sib=sib):
            eqp = iota == (pid - (c * CW).astype(jnp.float32))
            pm2 = jnp.where(eqp, px_ref[c], pm_ref[c])
            pa2 = jnp.where(eqp, sib, pa_ref[c])
            px_ref[c] = jnp.where(eqp, inf, px_ref[c])
            pm_ref[c] = pm2
            pa_ref[c] = pa2
            cm = jnp.min(pm2, axis=1, keepdims=True)
            cs = jnp.min(jnp.where(pm2 == cm, pa2, big), axis=1, keepdims=True)
            return merge(carry, cm, cs)

        m, s = lax.fori_loop(0, NCHP, scan_chunk, init)
        dist_ref[:, t:t + 1] = jnp.sqrt(jnp.maximum(m, 1e-12))
        idx_ref[:, t:t + 1] = s.astype(jnp.int32)
        sprev = s

    lv = lab[...]
    mx = jnp.max(lv, axis=1, keepdims=True)
    i13 = lax.broadcasted_iota(jnp.int32, lv.shape, 1)
    cl = jnp.min(jnp.where(lv == mx, i13, jnp.int32(C)), axis=1, keepdims=True)
    cls_ref[...] = cl.reshape(1, R, 1)


def _build_knn(interpret=False):
    return pl.pallas_call(
        _knn_body,
        grid=(GRID,),
        in_specs=[
            pl.BlockSpec((R, 3), lambda b: (b, 0)),
            pl.BlockSpec((NCH, 3, CW), lambda b: (0, 0, 0)),
            pl.BlockSpec((R, C), lambda b: (b, 0)),
            pl.BlockSpec((1, 1), lambda b: (0, 0)),
        ],
        out_specs=[
            pl.BlockSpec((R, K), lambda b: (b, 0)),
            pl.BlockSpec((R, K), lambda b: (b, 0)),
            pl.BlockSpec((1, R, 1), lambda b: (b, 0, 0)),
        ],
        out_shape=[
            jax.ShapeDtypeStruct((N, K), jnp.float32),
            jax.ShapeDtypeStruct((N, K), jnp.int32),
            jax.ShapeDtypeStruct((GRID, R, 1), jnp.int32),
        ],
        scratch_shapes=[
            pltpu.VMEM((NCHP, R, CW), jnp.float32),
            pltpu.VMEM((NCHP, R, CW), jnp.float32),
            pltpu.VMEM((NCHP, R, CW), jnp.float32),
        ],
        interpret=interpret,
    )


_knn = _build_knn()


def _sc_body(idxT_hbm, distT_hbm, cls_hbm, out_hbm, cls_v, idx_v, dist_v, out_v):
    wid = lax.axis_index("s") * _NCORES + lax.axis_index("c")
    base = wid * _PPW
    pltpu.sync_copy(cls_hbm, cls_v)
    pltpu.sync_copy(idxT_hbm.at[:, pl.ds(base, _PPW)], idx_v)
    pltpu.sync_copy(distT_hbm.at[:, pl.ds(base, _PPW)], dist_v)

    def grp(g, carry):
        own = cls_v[pl.ds(base + g * 16, 16)]
        cnt = jnp.zeros((16,), jnp.int32)
        wsum = jnp.zeros((16,), jnp.float32)
        wneg = jnp.zeros((16,), jnp.float32)
        for j in range(1, K):
            idxv = idx_v[j, pl.ds(g * 16, 16)]
            nlab = plsc.load_gather(cls_v, [idxv])
            dv = dist_v[j, pl.ds(g * 16, 16)]
            wv = jnp.exp(-dv)
            agree = nlab == own
            cnt = cnt + agree.astype(jnp.int32)
            wsum = wsum + wv
            wneg = wneg + jnp.where(agree, jnp.zeros((16,), jnp.float32), wv)
        amb = wneg / (wsum + 1e-8)
        res = jnp.where(cnt == 0, jnp.full((16,), 1.0, jnp.float32),
                        jnp.where(cnt == NS, jnp.zeros((16,), jnp.float32), amb))
        out_v[pl.ds(g * 16, 16)] = res
        return carry

    lax.fori_loop(0, _GPW, grp, 0)
    pltpu.sync_copy(out_v, out_hbm.at[pl.ds(base, _PPW)])


@functools.lru_cache(maxsize=1)
def _sc_ambiguity():
    return functools.partial(
        pl.kernel,
        mesh=plsc.VectorSubcoreMesh(core_axis_name="c", subcore_axis_name="s"),
        out_type=jax.ShapeDtypeStruct((N,), jnp.float32),
        scratch_types=[
            pltpu.VMEM((N,), jnp.int32),
            pltpu.VMEM((K, _PPW), jnp.int32),
            pltpu.VMEM((K, _PPW), jnp.float32),
            pltpu.VMEM((_PPW,), jnp.float32),
        ],
        compiler_params=pltpu.CompilerParams(needs_layout_passes=False),
    )(_sc_body)


def kernel(p, labels, nsample):
    delta = (jnp.asarray(nsample, jnp.float32) - jnp.float32(K)).reshape(1, 1)
    pTc = p.T.reshape(3, NCH, CW).transpose(1, 0, 2)
    dist, idx, cls3 = _knn(p, pTc, labels, delta)
    cls = cls3.reshape(N)
    return _sc_ambiguity()(idx.T, dist.T, cls)
